# baseline (speedup 1.0000x reference)
import jax
import jax.numpy as jnp
from jax.experimental import pallas as pl
from jax.experimental.pallas import tpu as pltpu

_BM = 8192
_BF = jnp.bfloat16


def _dott(a, w):
    return jax.lax.dot_general(a, w, (((1,), (1,)), ((), ())),
                               preferred_element_type=jnp.float32)


def _kernel(x_ref, w1, b1, w2, b2, w3, b3, w4, b4, w5r, b5, w6, b6,
            w7, b7, w8, b8, out_ref, acc_ref, c5_ref, w4b):
    ph = pl.program_id(0)
    j = pl.program_id(1)

    def h2_of_x():
        h = jax.lax.dot_general(x_ref[...], w1[...], (((0,), (1,)), ((), ())),
                                preferred_element_type=jnp.float32)
        h = jnp.maximum(h + b1[...], 0.0)
        h = jnp.maximum(_dott(h, w2[...]) + b2[...], 0.0)
        return h

    @pl.when((ph == 0) & (j == 0))
    def _cast_pool_weights():
        w4b[...] = w4[...].astype(_BF)

    @pl.when(ph == 0)
    def _pool():
        h = h2_of_x()
        y = jnp.maximum(
            jax.lax.dot_general(h.astype(_BF), w3[...].astype(_BF),
                                (((1,), (1,)), ((), ())),
                                preferred_element_type=jnp.float32)
            + b3[...], 0.0)
        y = _dott(y.astype(_BF), w4b[...])
        bm = jnp.max(y, axis=0, keepdims=True)

        @pl.when(j == 0)
        def _():
            acc_ref[...] = bm

        @pl.when(j > 0)
        def _():
            acc_ref[...] = jnp.maximum(acc_ref[...], bm)

    @pl.when((ph == 1) & (j == 0))
    def _globals():
        g = jnp.maximum(acc_ref[...] + b4[...], 0.0)
        c5_ref[...] = _dott(g, w5r[:, :1024]) + b5[...]

    @pl.when(ph == 1)
    def _head():
        h = h2_of_x()
        z = jnp.maximum(_dott(h, w5r[:, 1024:]) + c5_ref[...], 0.0)
        z = jnp.maximum(_dott(z, w6[...]) + b6[...], 0.0)
        z = jnp.maximum(_dott(z, w7[...]) + b7[...], 0.0)
        zw = (z * w8[...]).reshape(_BM // 128, 128, 128)
        out_ref[...] = jnp.sum(zw, axis=-1) + b8[0, 0]


def _full(shape):
    return pl.BlockSpec(shape, lambda i, j: tuple(0 for _ in shape))


def kernel(x, W1, b1, W2, b2, W3, b3, W4, b4, W5, b5, W6, b6, W7, b7, W8, b8):
    n = x.shape[2]
    pts_t = x.reshape(n, 3).T
    w5r = jnp.concatenate([W5[:, 64:], W5[:, :64]], axis=1)
    b1, b2, b3, b4 = (b.reshape(1, -1) for b in (b1, b2, b3, b4))
    b5, b6, b7, b8 = (b.reshape(1, -1) for b in (b5, b6, b7, b8))

    g2 = n // _BM
    out = pl.pallas_call(
        _kernel,
        grid=(2, g2),
        in_specs=[
            pl.BlockSpec((3, _BM), lambda i, j: (0, j)),
            _full((64, 3)), _full((1, 64)),
            _full((64, 64)), _full((1, 64)),
            _full((128, 64)), _full((1, 128)),
            _full((1024, 128)), _full((1, 1024)),
            _full((512, 1088)), _full((1, 512)),
            _full((256, 512)), _full((1, 256)),
            _full((128, 256)), _full((1, 128)),
            _full((1, 128)), pl.BlockSpec(memory_space=pltpu.SMEM),
        ],
        out_specs=pl.BlockSpec((_BM // 128, 128), lambda i, j: (i * j, 0)),
        out_shape=jax.ShapeDtypeStruct((n // 128, 128), jnp.float32),
        scratch_shapes=[
            pltpu.VMEM((1, 1024), jnp.float32),
            pltpu.VMEM((1, 512), jnp.float32),
            pltpu.VMEM((1024, 128), _BF),
        ],
        compiler_params=pltpu.CompilerParams(
            dimension_semantics=("arbitrary", "arbitrary"),
        ),
        name="pointnet_fused",
    )(pts_t, W1, b1, W2, b2, W3, b3, W4, b4, w5r, b5, W6, b6, W7, b7, W8, b8)

    return out.reshape(1, 1, n, 1)


# speedup vs baseline: 1.0044x; 1.0044x over previous
import jax
import jax.numpy as jnp
from jax.experimental import pallas as pl
from jax.experimental.pallas import tpu as pltpu

_BM = 8192
_BF = jnp.bfloat16


def _dott(a, w):
    return jax.lax.dot_general(a, w, (((1,), (1,)), ((), ())),
                               preferred_element_type=jnp.float32)


def _kernel(x_ref, w1, b1, w2, b2, w3, b3, w4, b4, w5r, b5, w6, b6,
            w7, b7, w8, b8, out_ref, acc_ref, c5_ref):
    ph = pl.program_id(0)
    j = pl.program_id(1)

    def h2_of_x():
        h = jax.lax.dot_general(x_ref[...], w1[...], (((0,), (1,)), ((), ())),
                                preferred_element_type=jnp.float32)
        h = jnp.maximum(h + b1[...], 0.0)
        h = jnp.maximum(_dott(h, w2[...]) + b2[...], 0.0)
        return h

    @pl.when(ph == 0)
    def _pool():
        h = h2_of_x()
        y = jnp.maximum(_dott(h, w3[...]) + b3[...], 0.0)
        y = _dott(y, w4[...])
        bm = jnp.max(y, axis=0, keepdims=True)

        @pl.when(j == 0)
        def _():
            acc_ref[...] = bm

        @pl.when(j > 0)
        def _():
            acc_ref[...] = jnp.maximum(acc_ref[...], bm)

    @pl.when((ph == 1) & (j == 0))
    def _globals():
        g = jnp.maximum(acc_ref[...] + b4[...], 0.0)
        c5_ref[...] = _dott(g, w5r[:, :1024]) + b5[...]

    @pl.when(ph == 1)
    def _head():
        h = h2_of_x()
        z = jnp.maximum(_dott(h, w5r[:, 1024:]) + c5_ref[...], 0.0)
        z = jnp.maximum(_dott(z, w6[...]) + b6[...], 0.0)
        z = jnp.maximum(_dott(z, w7[...]) + b7[...], 0.0)
        zw = (z * w8[...]).reshape(_BM // 128, 128, 128)
        out_ref[...] = jnp.sum(zw, axis=-1) + b8[0, 0]


def _full(shape):
    return pl.BlockSpec(shape, lambda i, j: tuple(0 for _ in shape))


def kernel(x, W1, b1, W2, b2, W3, b3, W4, b4, W5, b5, W6, b6, W7, b7, W8, b8):
    n = x.shape[2]
    pts_t = x.reshape(n, 3).T
    w5r = jnp.concatenate([W5[:, 64:], W5[:, :64]], axis=1)
    b1, b2, b3, b4 = (b.reshape(1, -1) for b in (b1, b2, b3, b4))
    b5, b6, b7, b8 = (b.reshape(1, -1) for b in (b5, b6, b7, b8))

    g2 = n // _BM
    out = pl.pallas_call(
        _kernel,
        grid=(2, g2),
        in_specs=[
            pl.BlockSpec((3, _BM), lambda i, j: (0, j)),
            _full((64, 3)), _full((1, 64)),
            _full((64, 64)), _full((1, 64)),
            _full((128, 64)), _full((1, 128)),
            _full((1024, 128)), _full((1, 1024)),
            _full((512, 1088)), _full((1, 512)),
            _full((256, 512)), _full((1, 256)),
            _full((128, 256)), _full((1, 128)),
            _full((1, 128)), pl.BlockSpec(memory_space=pltpu.SMEM),
        ],
        out_specs=pl.BlockSpec((_BM // 128, 128), lambda i, j: (i * j, 0)),
        out_shape=jax.ShapeDtypeStruct((n // 128, 128), jnp.float32),
        scratch_shapes=[
            pltpu.VMEM((1, 1024), jnp.float32),
            pltpu.VMEM((1, 512), jnp.float32),
        ],
        compiler_params=pltpu.CompilerParams(
            dimension_semantics=("arbitrary", "arbitrary"),
        ),
        name="pointnet_fused",
    )(pts_t, W1, b1, W2, b2, W3, b3, W4, b4, w5r, b5, W6, b6, W7, b7, W8, b8)

    return out.reshape(1, 1, n, 1)
